# Initial kernel scaffold; baseline (speedup 1.0000x reference)
import jax
import jax.numpy as jnp
from jax.experimental import pallas as pl


def kernel(input_text, text_len, emb_table, W1, b1, W2, b2):
    raise NotImplementedError("write your pallas kernel here")


# speedup vs baseline: 3.0533x; 3.0533x over previous
import functools

import jax
import jax.numpy as jnp
from jax import lax
from jax.experimental import pallas as pl
from jax.experimental.pallas import tpu as pltpu
from jax.experimental.pallas import tpu_sc as plsc

B, L, D = 16384, 200, 64
H, C = 50, 10
NC, NS = 2, 16
NW = NC * NS
RPW = B // NW
CHUNK = 100
CPR = L // CHUNK
GROUP = 64
NGROUPS = RPW // GROUP
NBUF = 2
NLANE = 16
DV = D // NLANE


def _sc_body(idx_hbm, table_hbm, out_hbm, idx_v, rows_v, out_v, sem0, sem1):
    wid = lax.axis_index("s") * NC + lax.axis_index("c")
    sems = (sem0, sem1)

    def fire(buf, row):
        for h in range(CPR):
            pltpu.make_async_copy(
                table_hbm.at[idx_v.at[row, h]],
                rows_v.at[buf, pl.ds(h * CHUNK, CHUNK), :],
                sems[buf],
            ).start()

    def drain(buf):
        for h in range(CPR):
            pltpu.make_async_copy(
                table_hbm.at[idx_v.at[0, 0]],
                rows_v.at[buf, pl.ds(h * CHUNK, CHUNK), :],
                sems[buf],
            ).wait()

    def accum_store(buf, out_row):
        rbuf = rows_v.at[buf]

        def it(i, acc):
            return tuple(acc[k] + rbuf[i, pl.ds(NLANE * k, NLANE)]
                         for k in range(DV))

        acc0 = tuple(jnp.zeros((NLANE,), jnp.float32) for _ in range(DV))
        acc = lax.fori_loop(0, L, it, acc0, unroll=8)
        for k in range(DV):
            out_v[out_row, pl.ds(NLANE * k, NLANE)] = acc[k]

    @pl.loop(0, NGROUPS)
    def _(g):
        pltpu.sync_copy(idx_hbm.at[wid, g], idx_v)
        for b in range(NBUF):
            fire(b, b)

        @pl.loop(0, GROUP, step=NBUF)
        def _(r0):
            for b in range(NBUF):
                r = r0 + b
                drain(b)
                accum_store(b, g * GROUP + r)
                nxt = r + NBUF

                @pl.when(nxt < GROUP)
                def _():
                    fire(b, nxt)

    pltpu.sync_copy(out_v, out_hbm.at[pl.ds(wid * RPW, RPW), :])


_sc_pool = functools.partial(
    pl.kernel,
    out_type=jax.ShapeDtypeStruct((B, D), jnp.float32),
    mesh=plsc.VectorSubcoreMesh(core_axis_name="c", subcore_axis_name="s",
                                num_cores=NC, num_subcores=NS),
    scratch_types=[
        pltpu.VMEM((GROUP, CPR, CHUNK), jnp.int32),
        pltpu.VMEM((NBUF, L, D), jnp.float32),
        pltpu.VMEM((RPW, D), jnp.float32),
        pltpu.SemaphoreType.DMA,
        pltpu.SemaphoreType.DMA,
    ],
    compiler_params=pltpu.CompilerParams(use_tc_tiling_on_sc=False),
)(_sc_body)


BM = 2048


def _mlp_body(x_ref, tl_ref, w1_ref, b1_ref, w2_ref, b2_ref, o_ref):
    x = x_ref[...] / tl_ref[...]
    h = jnp.maximum(
        jnp.dot(x, w1_ref[...], preferred_element_type=jnp.float32)
        + b1_ref[...], 0.0)
    o_ref[...] = (jnp.dot(h, w2_ref[...], preferred_element_type=jnp.float32)
                  + b2_ref[...])


def _mlp(pooled, text_len, W1, b1, W2, b2):
    return pl.pallas_call(
        _mlp_body,
        grid=(B // BM,),
        in_specs=[
            pl.BlockSpec((BM, D), lambda i: (i, 0)),
            pl.BlockSpec((BM, 1), lambda i: (i, 0)),
            pl.BlockSpec((D, H), lambda i: (0, 0)),
            pl.BlockSpec((1, H), lambda i: (0, 0)),
            pl.BlockSpec((H, C), lambda i: (0, 0)),
            pl.BlockSpec((1, C), lambda i: (0, 0)),
        ],
        out_specs=pl.BlockSpec((BM, C), lambda i: (i, 0)),
        out_shape=jax.ShapeDtypeStruct((B, C), jnp.float32),
    )(pooled, text_len.reshape(B, 1), W1, b1.reshape(1, H), W2,
      b2.reshape(1, C))


def kernel(input_text, text_len, emb_table, W1, b1, W2, b2):
    idx = input_text.astype(jnp.int32).reshape(NW, NGROUPS, GROUP, CPR, CHUNK)
    pooled = _sc_pool(idx, emb_table)
    return _mlp(pooled, text_len, W1, b1, W2, b2)
